# baseline (speedup 1.0000x reference)
import jax
import jax.numpy as jnp
from jax import lax
from jax.experimental import pallas as pl
from jax.experimental.pallas import tpu as pltpu

_LAT, _K = 64, 1024
_BLK = 448


def _conv2d(x, w, b, stride, pad):
    out = lax.conv_general_dilated(
        x, w, (stride, stride), [(pad, pad), (pad, pad)],
        dimension_numbers=('NCHW', 'OIHW', 'NCHW'))
    return out + b[None, :, None, None]


def _conv_t2d(x, w, b, stride, k):
    w2 = jnp.flip(w, axis=(2, 3)).transpose(1, 0, 2, 3)
    Bn, C, H, W = x.shape
    xd = jnp.zeros((Bn, C, (H - 1) * stride + 1, (W - 1) * stride + 1), x.dtype)
    xd = xd.at[:, :, ::stride, ::stride].set(x)
    out = lax.conv_general_dilated(
        xd, w2, (1, 1), [(k - 1, k - 1), (k - 1, k - 1)],
        dimension_numbers=('NCHW', 'OIHW', 'NCHW'))
    return out + b[None, :, None, None]


def _batchnorm(x, g, b, eps=1e-5):
    m = jnp.mean(x, axis=(0, 2, 3), keepdims=True)
    v = jnp.var(x, axis=(0, 2, 3), keepdims=True)
    return g[None, :, None, None] * (x - m) / jnp.sqrt(v + eps) + b[None, :, None, None]


def _leaky(x):
    return jnp.where(x >= 0, x, 0.01 * x)


def _vq_body(z_ref, cb_ref, idx_ref, quant_ref, rloss_ref):
    z = z_ref[...]
    cb = cb_ref[...]
    s = lax.dot_general(z, cb, (((1,), (1,)), ((), ())),
                        preferred_element_type=jnp.float32)
    zsq = jnp.sum(z * z, axis=1, keepdims=True)
    cbsq = jnp.sum(cb * cb, axis=1)
    d2 = zsq - 2.0 * s + cbsq[None, :]
    dist = jnp.sqrt(jnp.maximum(d2, 0.0))
    m = jnp.min(dist, axis=1, keepdims=True)
    ids = lax.broadcasted_iota(jnp.int32, (_BLK, _K), 1)
    idx = jnp.min(jnp.where(dist == m, ids, _K), axis=1)
    idx_ref[0, 0, :] = idx
    onehot = (ids == idx[:, None]).astype(jnp.float32)
    quant = lax.dot_general(onehot, cb, (((1,), (0,)), ((), ())),
                            preferred_element_type=jnp.float32)
    quant_ref[...] = quant
    r = z - quant
    rloss_ref[0, 0, :] = jnp.sum(r * r, axis=1)


def _vq(z, cb):
    rows = z.shape[0]
    nblk = rows // _BLK
    idx3, quant, rloss = pl.pallas_call(
        _vq_body,
        grid=(nblk,),
        in_specs=[
            pl.BlockSpec((_BLK, _LAT), lambda i: (i, 0)),
            pl.BlockSpec((_K, _LAT), lambda i: (0, 0)),
        ],
        out_specs=[
            pl.BlockSpec((1, 1, _BLK), lambda i: (i, 0, 0)),
            pl.BlockSpec((_BLK, _LAT), lambda i: (i, 0)),
            pl.BlockSpec((1, 1, _BLK), lambda i: (i, 0, 0)),
        ],
        out_shape=[
            jax.ShapeDtypeStruct((nblk, 1, _BLK), jnp.int32),
            jax.ShapeDtypeStruct((rows, _LAT), jnp.float32),
            jax.ShapeDtypeStruct((nblk, 1, _BLK), jnp.float32),
        ],
    )(z, cb)
    idx = idx3.reshape(rows)
    loss = jnp.sum(rloss) / (rows * _LAT)
    return idx, quant, loss


def kernel(x, params):
    p = params
    out = _leaky(_batchnorm(_conv2d(x, p['enc_w0'], p['enc_b0'], 2, 1),
                            p['enc_g0'], p['enc_be0']))
    out = _leaky(_batchnorm(_conv2d(out, p['enc_w1'], p['enc_b1'], 2, 1),
                            p['enc_g1'], p['enc_be1']))
    out = _conv2d(out, p['enc_w2'], p['enc_b2'], 2, 1)
    out = _conv2d(out, p['preq_w'], p['preq_b'], 1, 0)
    Bn, lat, H, W = out.shape
    z = out.transpose(0, 2, 3, 1).reshape(Bn * H * W, lat)
    idx, quant, loss = _vq(z, p['codebook'])
    idx = idx.reshape(Bn, H, W)
    quant = quant.reshape(Bn, H, W, lat).transpose(0, 3, 1, 2)
    out = _conv2d(quant, p['postq_w'], p['postq_b'], 1, 0)
    out = _leaky(_batchnorm(_conv_t2d(out, p['dec_w0'], p['dec_b0'], 2, 4),
                            p['dec_g0'], p['dec_be0']))
    out = _leaky(_batchnorm(_conv_t2d(out, p['dec_w1'], p['dec_b1'], 2, 4),
                            p['dec_g1'], p['dec_be1']))
    out = jnp.tanh(_conv_t2d(out, p['dec_w2'], p['dec_b2'], 2, 4))
    return (out, idx, loss, loss)


# speedup vs baseline: 9.8502x; 9.8502x over previous
import jax
import jax.numpy as jnp
from jax import lax
from jax.experimental import pallas as pl
from jax.experimental.pallas import tpu as pltpu

_LAT, _K = 64, 1024
_BLK = 448


def _conv2d(x, w, b, stride, pad):
    out = lax.conv_general_dilated(
        x, w, (stride, stride), [(pad, pad), (pad, pad)],
        dimension_numbers=('NCHW', 'OIHW', 'NCHW'))
    return out + b[None, :, None, None]


def _conv_t2d(x, w, b, stride, k):
    w2 = jnp.flip(w, axis=(2, 3)).transpose(1, 0, 2, 3)
    out = lax.conv_general_dilated(
        x, w2, (1, 1), [(k - 1, k - 1), (k - 1, k - 1)],
        lhs_dilation=(stride, stride),
        dimension_numbers=('NCHW', 'OIHW', 'NCHW'))
    return out + b[None, :, None, None]


def _batchnorm(x, g, b, eps=1e-5):
    m = jnp.mean(x, axis=(0, 2, 3), keepdims=True)
    v = jnp.var(x, axis=(0, 2, 3), keepdims=True)
    return g[None, :, None, None] * (x - m) / jnp.sqrt(v + eps) + b[None, :, None, None]


def _leaky(x):
    return jnp.where(x >= 0, x, 0.01 * x)


def _vq_body(z_ref, cb_ref, idx_ref, quant_ref, rloss_ref):
    z = z_ref[...]
    cb = cb_ref[...]
    s = lax.dot_general(z, cb, (((1,), (1,)), ((), ())),
                        preferred_element_type=jnp.float32)
    zsq = jnp.sum(z * z, axis=1, keepdims=True)
    cbsq = jnp.sum(cb * cb, axis=1)
    d2 = zsq - 2.0 * s + cbsq[None, :]
    dist = jnp.sqrt(jnp.maximum(d2, 0.0))
    m = jnp.min(dist, axis=1, keepdims=True)
    ids = lax.broadcasted_iota(jnp.int32, (_BLK, _K), 1)
    idx = jnp.min(jnp.where(dist == m, ids, _K), axis=1)
    idx_ref[0, 0, :] = idx
    onehot = (ids == idx[:, None]).astype(jnp.float32)
    quant = lax.dot_general(onehot, cb, (((1,), (0,)), ((), ())),
                            preferred_element_type=jnp.float32)
    quant_ref[...] = quant
    r = z - quant
    rloss_ref[0, 0, :] = jnp.sum(r * r, axis=1)


def _vq(z, cb):
    rows = z.shape[0]
    nblk = rows // _BLK
    idx3, quant, rloss = pl.pallas_call(
        _vq_body,
        grid=(nblk,),
        in_specs=[
            pl.BlockSpec((_BLK, _LAT), lambda i: (i, 0)),
            pl.BlockSpec((_K, _LAT), lambda i: (0, 0)),
        ],
        out_specs=[
            pl.BlockSpec((1, 1, _BLK), lambda i: (i, 0, 0)),
            pl.BlockSpec((_BLK, _LAT), lambda i: (i, 0)),
            pl.BlockSpec((1, 1, _BLK), lambda i: (i, 0, 0)),
        ],
        out_shape=[
            jax.ShapeDtypeStruct((nblk, 1, _BLK), jnp.int32),
            jax.ShapeDtypeStruct((rows, _LAT), jnp.float32),
            jax.ShapeDtypeStruct((nblk, 1, _BLK), jnp.float32),
        ],
    )(z, cb)
    idx = idx3.reshape(rows)
    loss = jnp.sum(rloss) / (rows * _LAT)
    return idx, quant, loss


def kernel(x, params):
    p = params
    out = _leaky(_batchnorm(_conv2d(x, p['enc_w0'], p['enc_b0'], 2, 1),
                            p['enc_g0'], p['enc_be0']))
    out = _leaky(_batchnorm(_conv2d(out, p['enc_w1'], p['enc_b1'], 2, 1),
                            p['enc_g1'], p['enc_be1']))
    out = _conv2d(out, p['enc_w2'], p['enc_b2'], 2, 1)
    out = _conv2d(out, p['preq_w'], p['preq_b'], 1, 0)
    Bn, lat, H, W = out.shape
    z = out.transpose(0, 2, 3, 1).reshape(Bn * H * W, lat)
    idx, quant, loss = _vq(z, p['codebook'])
    idx = idx.reshape(Bn, H, W)
    quant = quant.reshape(Bn, H, W, lat).transpose(0, 3, 1, 2)
    out = _conv2d(quant, p['postq_w'], p['postq_b'], 1, 0)
    out = _leaky(_batchnorm(_conv_t2d(out, p['dec_w0'], p['dec_b0'], 2, 4),
                            p['dec_g0'], p['dec_be0']))
    out = _leaky(_batchnorm(_conv_t2d(out, p['dec_w1'], p['dec_b1'], 2, 4),
                            p['dec_g1'], p['dec_be1']))
    out = jnp.tanh(_conv_t2d(out, p['dec_w2'], p['dec_b2'], 2, 4))
    return (out, idx, loss, loss)
